# baseline (speedup 1.0000x reference)
import functools

import jax
import jax.numpy as jnp
from jax import lax
from jax.experimental import pallas as pl
from jax.experimental.pallas import tpu as pltpu
from jax.experimental.pallas import tpu_sc as plsc

_NC = 2
_NS = 16
_NW = _NC * _NS


def _sc_gather(idx2, types, emb):
    R = idx2.shape[0]
    D = emb.shape[1]
    PL = R * 128
    rows_per_tile = R // _NW

    mesh = plsc.VectorSubcoreMesh(core_axis_name="c", subcore_axis_name="s")

    @functools.partial(
        pl.kernel,
        out_type=jax.ShapeDtypeStruct((PL, D), jnp.float32),
        mesh=mesh,
        scratch_types=[
            pltpu.VMEM((rows_per_tile, 128), jnp.int32),
            pltpu.VMEM((2, 128), jnp.int32),
            pltpu.VMEM((2, 128, D), jnp.float32),
            pltpu.SemaphoreType.DMA,
            pltpu.SemaphoreType.DMA,
            pltpu.SemaphoreType.DMA,
        ],
    )
    def k(idx_hbm, types_hbm, emb_hbm, out_hbm, idx_v, t_v, rows_v, sem1, sem2,
          sem3):
        wid = lax.axis_index("s") * _NC + lax.axis_index("c")
        base_row = wid * rows_per_tile
        pltpu.sync_copy(idx_hbm.at[pl.ds(base_row, rows_per_tile)], idx_v)
        pltpu.async_copy(types_hbm.at[idx_v.at[0]], t_v.at[0], sem1)
        for j in range(rows_per_tile):
            b = j % 2
            pltpu.make_async_copy(types_hbm.at[idx_v.at[j]], t_v.at[b],
                                  sem1).wait()
            if j + 1 < rows_per_tile:
                pltpu.async_copy(types_hbm.at[idx_v.at[j + 1]], t_v.at[1 - b],
                                 sem1)
            if j >= 2:
                pltpu.make_async_copy(
                    rows_v.at[b], out_hbm.at[pl.ds((base_row + j - 2) * 128,
                                                   128)], sem3).wait()
            pltpu.async_copy(emb_hbm.at[t_v.at[b]], rows_v.at[b], sem2)
            pltpu.make_async_copy(emb_hbm.at[t_v.at[b]], rows_v.at[b],
                                  sem2).wait()
            pltpu.async_copy(rows_v.at[b],
                             out_hbm.at[pl.ds((base_row + j) * 128, 128)], sem3)
        for j in range(rows_per_tile - 2, rows_per_tile):
            b = j % 2
            pltpu.make_async_copy(
                rows_v.at[b], out_hbm.at[pl.ds((base_row + j) * 128, 128)],
                sem3).wait()

    return k(idx2, types, emb)


def _tc_encode(occ3, child, vert, maskf, orient_emb, proj_W, proj_b2,
               gru_Wih, gru_Whh, gru_bih2, gru_bhh2):
    P, L, D = occ3.shape
    BP = 512
    grid = (P // BP,)

    def body(occ_ref, child_ref, vert_ref, mask_ref, oemb_ref, projW_ref,
             projb_ref, wih_ref, whh_ref, bih_ref, bhh_ref,
             nodes_out_ref, orient_out_ref):
        oemb = oemb_ref[...]
        wih = wih_ref[...]
        Ao = jnp.dot(jnp.dot(oemb, projW_ref[:D, :],
                             preferred_element_type=jnp.float32), wih,
                     preferred_element_type=jnp.float32)
        Bo = jnp.dot(jnp.dot(oemb, projW_ref[D:, :],
                             preferred_element_type=jnp.float32), wih,
                     preferred_element_type=jnp.float32)
        pbW = jnp.dot(projb_ref[...], wih, preferred_element_type=jnp.float32)
        bih = bih_ref[...]
        whh = whh_ref[...]
        bhh = bhh_ref[...]
        iota = lax.broadcasted_iota(jnp.int32, (1, oemb.shape[0]), 1)

        def gru_step(gx, h):
            gh = jnp.dot(h, whh, preferred_element_type=jnp.float32) + bhh
            z = jax.nn.sigmoid(gx[:, :D] + gh[:, :D])
            r = jax.nn.sigmoid(gx[:, D:2 * D] + gh[:, D:2 * D])
            n = jnp.tanh(gx[:, 2 * D:] + r * gh[:, 2 * D:])
            return (1.0 - z) * n + z * h

        h = jnp.zeros((BP, D), jnp.float32)
        for l in range(L):
            ml = mask_ref[:, l:l + 1]
            gxe = ml * jnp.dot(occ_ref[:, l, :], wih,
                               preferred_element_type=jnp.float32) + bih
            h = gru_step(gxe, h)
            nodes_out_ref[:, l, :] = h * ml
            ohc = (child_ref[:, l:l + 1] == iota).astype(jnp.float32)
            ohv = (vert_ref[:, l:l + 1] == iota).astype(jnp.float32)
            gxo = ml * (jnp.dot(ohc, Ao, preferred_element_type=jnp.float32)
                        + jnp.dot(ohv, Bo, preferred_element_type=jnp.float32)
                        + pbW) + bih
            h = gru_step(gxo, h)
            orient_out_ref[:, l, :] = h * ml

    V = orient_emb.shape[0]
    out = pl.pallas_call(
        body,
        grid=grid,
        in_specs=[
            pl.BlockSpec((BP, L, D), lambda i: (i, 0, 0)),
            pl.BlockSpec((BP, L), lambda i: (i, 0)),
            pl.BlockSpec((BP, L), lambda i: (i, 0)),
            pl.BlockSpec((BP, L), lambda i: (i, 0)),
            pl.BlockSpec((V, D), lambda i: (0, 0)),
            pl.BlockSpec((2 * D, D), lambda i: (0, 0)),
            pl.BlockSpec((1, D), lambda i: (0, 0)),
            pl.BlockSpec((D, 3 * D), lambda i: (0, 0)),
            pl.BlockSpec((D, 3 * D), lambda i: (0, 0)),
            pl.BlockSpec((1, 3 * D), lambda i: (0, 0)),
            pl.BlockSpec((1, 3 * D), lambda i: (0, 0)),
        ],
        out_specs=[
            pl.BlockSpec((BP, L, D), lambda i: (i, 0, 0)),
            pl.BlockSpec((BP, L, D), lambda i: (i, 0, 0)),
        ],
        out_shape=[
            jax.ShapeDtypeStruct((P, L, D), jnp.float32),
            jax.ShapeDtypeStruct((P, L, D), jnp.float32),
        ],
    )(occ3, child, vert, maskf, orient_emb, proj_W, proj_b2,
      gru_Wih, gru_Whh, gru_bih2, gru_bhh2)
    return out


def _sc_scatter(contrib, idx64, n_nodes, d):
    R = idx64.shape[0]
    IW = idx64.shape[1]
    RANGE = 12512
    SP_ROWS = RANGE + 32
    NHB = 2
    irpt = R // _NS
    irph = irpt // NHB
    HB = irph * IW
    ZCH = SP_ROWS // 16
    WCH = RANGE // 16

    mesh = plsc.VectorSubcoreMesh(core_axis_name="c", subcore_axis_name="s")

    @functools.partial(
        pl.kernel,
        out_type=jax.ShapeDtypeStruct((n_nodes, d), jnp.float32),
        mesh=mesh,
        compiler_params=pltpu.CompilerParams(needs_layout_passes=False),
        scratch_types=[
            pltpu.VMEM_SHARED((SP_ROWS, d), jnp.float32),
            pltpu.VMEM((8, d), jnp.float32),
            pltpu.VMEM((2, IW), jnp.int32),
            pltpu.VMEM((16,), jnp.int32),
            pltpu.VMEM((HB + 16,), jnp.int32),
            pltpu.VMEM((2, 16, d), jnp.float32),
            pltpu.SemaphoreType.DMA,
            pltpu.SemaphoreType.DMA,
        ],
    )
    def k(contrib_hbm, idx_hbm, out_hbm, spmem, zbuf, ibuf, cntb, posb,
          cbuf, semr, semi):
        core = lax.axis_index("c")
        sub = lax.axis_index("s")
        trash = RANGE + sub
        zero16 = jnp.zeros((16,), jnp.float32)
        iota16 = lax.broadcasted_iota(jnp.int32, (16,), 0)
        for zr in range(8):
            for zc in range(d // 16):
                zbuf[zr, pl.ds(zc * 16, 16)] = zero16

        for p in range(2):
            lo = (2 * core + p) * RANGE

            def zero_body(kk):
                cid = kk * _NS + sub
                @pl.when(cid < SP_ROWS // 8)
                def _():
                    pltpu.sync_copy(zbuf, spmem.at[pl.ds(cid * 8, 8)])
            pl.loop(0, (SP_ROWS // 8 + _NS - 1) // _NS)(zero_body)
            plsc.subcore_barrier()

            for hb in range(NHB):
                row0 = sub * irpt + hb * irph
                base = row0 * IW

                pltpu.async_copy(idx_hbm.at[row0], ibuf.at[0], semi)

                def comp_body(j, cnt):
                    for b in range(2):
                        jj = 2 * j + b
                        pltpu.make_async_copy(idx_hbm.at[row0 + jj],
                                              ibuf.at[b], semi).wait()
                        @pl.when(jj + 1 < irph)
                        def _():
                            pltpu.async_copy(idx_hbm.at[row0 + jj + 1],
                                             ibuf.at[1 - b], semi)
                        for kk in range(IW // 16):
                            v = ibuf[b, pl.ds(kk * 16, 16)]
                            inr = (v >= lo) & (v < lo + RANGE)
                            pos = base + jj * IW + kk * 16 + iota16
                            key = jnp.where(inr, 0, 1).astype(jnp.uint32)
                            comb = pos | ((v - lo) << 17)
                            _, sc_ = plsc.sort_key_val(key, comb)
                            posb[pl.ds(cnt, 16)] = sc_
                            cnt = cnt + (
                                plsc.all_reduce_population_count(inr)[0])
                    return cnt
                cnt = lax.fori_loop(0, irph // 2, comp_body, 0)
                posb[pl.ds(cnt, 16)] = jnp.full((16,), base, jnp.int32) | (
                    trash << 17)
                nch = (cnt + 15) >> 4

                @pl.when(nch > 0)
                def _():
                    vp0 = posb[pl.ds(0, 16)] & 0x1FFFF
                    pltpu.async_copy(contrib_hbm.at[vp0], cbuf.at[0], semr)

                def sc_body(c, _):
                    for b in range(2):
                        cc = 2 * c + b
                        @pl.when(cc < nch)
                        def _():
                            vc = posb[pl.ds(cc * 16, 16)]
                            vp = vc & 0x1FFFF
                            @pl.when(cc + 1 < nch)
                            def _():
                                vpn = posb[pl.ds((cc + 1) * 16, 16)] & 0x1FFFF
                                pltpu.async_copy(contrib_hbm.at[vpn],
                                                 cbuf.at[1 - b], semr)
                            pltpu.make_async_copy(contrib_hbm.at[vp],
                                                  cbuf.at[b], semr).wait()
                            vt = vc >> 17
                            pltpu.sync_copy(cbuf.at[b], spmem.at[vt], add=True)
                    return 0
                lax.fori_loop(0, (nch + 1) >> 1, sc_body, 0)
            plsc.subcore_barrier()

            def wr_body(kk):
                cid = kk * _NS + sub
                s = lo + cid * 16
                @pl.when((cid < WCH) & (s < n_nodes))
                def _():
                    pltpu.sync_copy(spmem.at[pl.ds(cid * 16, 16)],
                                    out_hbm.at[pl.ds(s, 16)])
            pl.loop(0, (WCH + _NS - 1) // _NS)(wr_body)
            plsc.subcore_barrier()

    return k(contrib, idx64)


def kernel(ast_paths_node_indices, ast_paths_lengths, ast_paths_mask,
           ast_nodes_types, ast_paths_child_place, ast_paths_vertical_direction,
           node_type_emb, orient_emb, proj_W, proj_b,
           gru_Wih, gru_Whh, gru_bih, gru_bhh):
    P, L = ast_paths_node_indices.shape
    D = node_type_emb.shape[1]
    N = ast_nodes_types.shape[0]

    idx2 = ast_paths_node_indices.reshape(P * L // 128, 128).astype(jnp.int32)
    types = ast_nodes_types.astype(jnp.int32)
    maskf = ast_paths_mask.astype(jnp.float32)

    node_occ = _sc_gather(idx2, types, node_type_emb)
    occ3 = node_occ.reshape(P, L, D)

    nodes_enc, orient_enc = _tc_encode(
        occ3, ast_paths_child_place.astype(jnp.int32),
        ast_paths_vertical_direction.astype(jnp.int32), maskf,
        orient_emb, proj_W, proj_b.reshape(1, D),
        gru_Wih, gru_Whh, gru_bih.reshape(1, 3 * D), gru_bhh.reshape(1, 3 * D))

    contrib = nodes_enc.reshape(P * L, D)
    idx64 = ast_paths_node_indices.reshape(P * L // 32, 32).astype(jnp.int32)
    node_repr = _sc_scatter(contrib, idx64, N, D)
    return node_repr, nodes_enc, orient_enc


# speedup vs baseline: 1.0543x; 1.0543x over previous
import functools

import jax
import jax.numpy as jnp
from jax import lax
from jax.experimental import pallas as pl
from jax.experimental.pallas import tpu as pltpu
from jax.experimental.pallas import tpu_sc as plsc

_NC = 2
_NS = 16
_NW = _NC * _NS


def _sc_gather(idx2, types, emb):
    R = idx2.shape[0]
    D = emb.shape[1]
    PL = R * 128
    rows_per_tile = R // _NW

    mesh = plsc.VectorSubcoreMesh(core_axis_name="c", subcore_axis_name="s")

    @functools.partial(
        pl.kernel,
        out_type=jax.ShapeDtypeStruct((PL, D), jnp.float32),
        mesh=mesh,
        scratch_types=[
            pltpu.VMEM((rows_per_tile, 128), jnp.int32),
            pltpu.VMEM((2, 128), jnp.int32),
            pltpu.VMEM((2, 128, D), jnp.float32),
            pltpu.SemaphoreType.DMA,
            pltpu.SemaphoreType.DMA,
            pltpu.SemaphoreType.DMA,
        ],
    )
    def k(idx_hbm, types_hbm, emb_hbm, out_hbm, idx_v, t_v, rows_v, sem1, sem2,
          sem3):
        wid = lax.axis_index("s") * _NC + lax.axis_index("c")
        base_row = wid * rows_per_tile
        pltpu.sync_copy(idx_hbm.at[pl.ds(base_row, rows_per_tile)], idx_v)
        pltpu.async_copy(types_hbm.at[idx_v.at[0]], t_v.at[0], sem1)
        for j in range(rows_per_tile):
            b = j % 2
            pltpu.make_async_copy(types_hbm.at[idx_v.at[j]], t_v.at[b],
                                  sem1).wait()
            if j + 1 < rows_per_tile:
                pltpu.async_copy(types_hbm.at[idx_v.at[j + 1]], t_v.at[1 - b],
                                 sem1)
            if j >= 2:
                pltpu.make_async_copy(
                    rows_v.at[b], out_hbm.at[pl.ds((base_row + j - 2) * 128,
                                                   128)], sem3).wait()
            pltpu.async_copy(emb_hbm.at[t_v.at[b]], rows_v.at[b], sem2)
            pltpu.make_async_copy(emb_hbm.at[t_v.at[b]], rows_v.at[b],
                                  sem2).wait()
            pltpu.async_copy(rows_v.at[b],
                             out_hbm.at[pl.ds((base_row + j) * 128, 128)], sem3)
        for j in range(rows_per_tile - 2, rows_per_tile):
            b = j % 2
            pltpu.make_async_copy(
                rows_v.at[b], out_hbm.at[pl.ds((base_row + j) * 128, 128)],
                sem3).wait()

    return k(idx2, types, emb)


def _tc_encode(occ3, child, vert, maskf, orient_emb, proj_W, proj_b2,
               gru_Wih, gru_Whh, gru_bih2, gru_bhh2):
    P, L, D = occ3.shape
    BP = 512
    grid = (P // BP,)

    def body(occ_ref, child_ref, vert_ref, mask_ref, oemb_ref, projW_ref,
             projb_ref, wih_ref, whh_ref, bih_ref, bhh_ref,
             nodes_out_ref, orient_out_ref):
        oemb = oemb_ref[...]
        wih = wih_ref[...]
        Ao = jnp.dot(jnp.dot(oemb, projW_ref[:D, :],
                             preferred_element_type=jnp.float32), wih,
                     preferred_element_type=jnp.float32)
        Bo = jnp.dot(jnp.dot(oemb, projW_ref[D:, :],
                             preferred_element_type=jnp.float32), wih,
                     preferred_element_type=jnp.float32)
        pbW = jnp.dot(projb_ref[...], wih, preferred_element_type=jnp.float32)
        bih = bih_ref[...]
        whh = whh_ref[...]
        bhh = bhh_ref[...]
        iota = lax.broadcasted_iota(jnp.int32, (1, oemb.shape[0]), 1)

        def gru_step(gx, h):
            gh = jnp.dot(h, whh, preferred_element_type=jnp.float32) + bhh
            z = jax.nn.sigmoid(gx[:, :D] + gh[:, :D])
            r = jax.nn.sigmoid(gx[:, D:2 * D] + gh[:, D:2 * D])
            n = jnp.tanh(gx[:, 2 * D:] + r * gh[:, 2 * D:])
            return (1.0 - z) * n + z * h

        h = jnp.zeros((BP, D), jnp.float32)
        for l in range(L):
            ml = mask_ref[:, l:l + 1]
            gxe = ml * jnp.dot(occ_ref[:, l, :], wih,
                               preferred_element_type=jnp.float32) + bih
            h = gru_step(gxe, h)
            nodes_out_ref[:, l, :] = h * ml
            ohc = (child_ref[:, l:l + 1] == iota).astype(jnp.float32)
            ohv = (vert_ref[:, l:l + 1] == iota).astype(jnp.float32)
            gxo = ml * (jnp.dot(ohc, Ao, preferred_element_type=jnp.float32)
                        + jnp.dot(ohv, Bo, preferred_element_type=jnp.float32)
                        + pbW) + bih
            h = gru_step(gxo, h)
            orient_out_ref[:, l, :] = h * ml

    V = orient_emb.shape[0]
    out = pl.pallas_call(
        body,
        grid=grid,
        in_specs=[
            pl.BlockSpec((BP, L, D), lambda i: (i, 0, 0)),
            pl.BlockSpec((BP, L), lambda i: (i, 0)),
            pl.BlockSpec((BP, L), lambda i: (i, 0)),
            pl.BlockSpec((BP, L), lambda i: (i, 0)),
            pl.BlockSpec((V, D), lambda i: (0, 0)),
            pl.BlockSpec((2 * D, D), lambda i: (0, 0)),
            pl.BlockSpec((1, D), lambda i: (0, 0)),
            pl.BlockSpec((D, 3 * D), lambda i: (0, 0)),
            pl.BlockSpec((D, 3 * D), lambda i: (0, 0)),
            pl.BlockSpec((1, 3 * D), lambda i: (0, 0)),
            pl.BlockSpec((1, 3 * D), lambda i: (0, 0)),
        ],
        out_specs=[
            pl.BlockSpec((BP, L, D), lambda i: (i, 0, 0)),
            pl.BlockSpec((BP, L, D), lambda i: (i, 0, 0)),
        ],
        out_shape=[
            jax.ShapeDtypeStruct((P, L, D), jnp.float32),
            jax.ShapeDtypeStruct((P, L, D), jnp.float32),
        ],
    )(occ3, child, vert, maskf, orient_emb, proj_W, proj_b2,
      gru_Wih, gru_Whh, gru_bih2, gru_bhh2)
    return out


def _sc_scatter(contrib, idx64, n_nodes, d):
    R = idx64.shape[0]
    IW = idx64.shape[1]
    RANGE = 12512
    SP_ROWS = RANGE + 32
    NHB = 2
    irpt = R // _NS
    irph = irpt // NHB
    HB = irph * IW
    ZCH = SP_ROWS // 16
    WCH = RANGE // 16

    mesh = plsc.VectorSubcoreMesh(core_axis_name="c", subcore_axis_name="s")

    @functools.partial(
        pl.kernel,
        out_type=jax.ShapeDtypeStruct((n_nodes, d), jnp.float32),
        mesh=mesh,
        compiler_params=pltpu.CompilerParams(needs_layout_passes=False),
        scratch_types=[
            pltpu.VMEM_SHARED((SP_ROWS, d), jnp.float32),
            pltpu.VMEM((8, d), jnp.float32),
            pltpu.VMEM((2, IW), jnp.int32),
            pltpu.VMEM((16,), jnp.int32),
            pltpu.VMEM((HB + 16,), jnp.int32),
            pltpu.VMEM((4, 16, d), jnp.float32),
            pltpu.SemaphoreType.DMA,
            pltpu.SemaphoreType.DMA,
        ],
    )
    def k(contrib_hbm, idx_hbm, out_hbm, spmem, zbuf, ibuf, cntb, posb,
          cbuf, semr, semi):
        core = lax.axis_index("c")
        sub = lax.axis_index("s")
        trash = RANGE + sub
        zero16 = jnp.zeros((16,), jnp.float32)
        iota16 = lax.broadcasted_iota(jnp.int32, (16,), 0)
        for zr in range(8):
            for zc in range(d // 16):
                zbuf[zr, pl.ds(zc * 16, 16)] = zero16

        for p in range(2):
            lo = (2 * core + p) * RANGE

            def zero_body(kk):
                cid = kk * _NS + sub
                @pl.when(cid < SP_ROWS // 8)
                def _():
                    pltpu.sync_copy(zbuf, spmem.at[pl.ds(cid * 8, 8)])
            pl.loop(0, (SP_ROWS // 8 + _NS - 1) // _NS)(zero_body)
            plsc.subcore_barrier()

            for hb in range(NHB):
                row0 = sub * irpt + hb * irph
                base = row0 * IW

                pltpu.async_copy(idx_hbm.at[row0], ibuf.at[0], semi)

                def comp_body(j, cnt):
                    for b in range(2):
                        jj = 2 * j + b
                        pltpu.make_async_copy(idx_hbm.at[row0 + jj],
                                              ibuf.at[b], semi).wait()
                        @pl.when(jj + 1 < irph)
                        def _():
                            pltpu.async_copy(idx_hbm.at[row0 + jj + 1],
                                             ibuf.at[1 - b], semi)
                        for kk in range(IW // 16):
                            v = ibuf[b, pl.ds(kk * 16, 16)]
                            inr = (v >= lo) & (v < lo + RANGE)
                            pos = base + jj * IW + kk * 16 + iota16
                            key = jnp.where(inr, 0, 1).astype(jnp.uint32)
                            comb = pos | ((v - lo) << 17)
                            _, sc_ = plsc.sort_key_val(key, comb)
                            posb[pl.ds(cnt, 16)] = sc_
                            cnt = cnt + (
                                plsc.all_reduce_population_count(inr)[0])
                    return cnt
                cnt = lax.fori_loop(0, irph // 2, comp_body, 0)
                posb[pl.ds(cnt, 16)] = jnp.full((16,), base, jnp.int32) | (
                    trash << 17)
                nch = (cnt + 15) >> 4

                for q in range(3):
                    @pl.when(nch > q)
                    def _():
                        vpq = posb[pl.ds(q * 16, 16)] & 0x1FFFF
                        pltpu.async_copy(contrib_hbm.at[vpq], cbuf.at[q], semr)

                def sc_body(c, _):
                    for b in range(4):
                        cc = 4 * c + b
                        @pl.when(cc < nch)
                        def _():
                            vc = posb[pl.ds(cc * 16, 16)]
                            vp = vc & 0x1FFFF
                            @pl.when(cc + 3 < nch)
                            def _():
                                vpn = posb[pl.ds((cc + 3) * 16, 16)] & 0x1FFFF
                                pltpu.async_copy(contrib_hbm.at[vpn],
                                                 cbuf.at[(b + 3) % 4], semr)
                            pltpu.make_async_copy(contrib_hbm.at[vp],
                                                  cbuf.at[b], semr).wait()
                            vt = vc >> 17
                            pltpu.sync_copy(cbuf.at[b], spmem.at[vt], add=True)
                    return 0
                lax.fori_loop(0, (nch + 3) >> 2, sc_body, 0)
            plsc.subcore_barrier()

            def wr_body(kk):
                cid = kk * _NS + sub
                s = lo + cid * 16
                @pl.when((cid < WCH) & (s < n_nodes))
                def _():
                    pltpu.sync_copy(spmem.at[pl.ds(cid * 16, 16)],
                                    out_hbm.at[pl.ds(s, 16)])
            pl.loop(0, (WCH + _NS - 1) // _NS)(wr_body)
            plsc.subcore_barrier()

    return k(contrib, idx64)


def kernel(ast_paths_node_indices, ast_paths_lengths, ast_paths_mask,
           ast_nodes_types, ast_paths_child_place, ast_paths_vertical_direction,
           node_type_emb, orient_emb, proj_W, proj_b,
           gru_Wih, gru_Whh, gru_bih, gru_bhh):
    P, L = ast_paths_node_indices.shape
    D = node_type_emb.shape[1]
    N = ast_nodes_types.shape[0]

    idx2 = ast_paths_node_indices.reshape(P * L // 128, 128).astype(jnp.int32)
    types = ast_nodes_types.astype(jnp.int32)
    maskf = ast_paths_mask.astype(jnp.float32)

    node_occ = _sc_gather(idx2, types, node_type_emb)
    occ3 = node_occ.reshape(P, L, D)

    nodes_enc, orient_enc = _tc_encode(
        occ3, ast_paths_child_place.astype(jnp.int32),
        ast_paths_vertical_direction.astype(jnp.int32), maskf,
        orient_emb, proj_W, proj_b.reshape(1, D),
        gru_Wih, gru_Whh, gru_bih.reshape(1, 3 * D), gru_bhh.reshape(1, 3 * D))

    contrib = nodes_enc.reshape(P * L, D)
    idx64 = ast_paths_node_indices.reshape(P * L // 32, 32).astype(jnp.int32)
    node_repr = _sc_scatter(contrib, idx64, N, D)
    return node_repr, nodes_enc, orient_enc


# speedup vs baseline: 1.1258x; 1.0679x over previous
import functools

import jax
import jax.numpy as jnp
from jax import lax
from jax.experimental import pallas as pl
from jax.experimental.pallas import tpu as pltpu
from jax.experimental.pallas import tpu_sc as plsc

_NC = 2
_NS = 16
_NW = _NC * _NS


def _sc_gather(idx2, types, emb):
    R = idx2.shape[0]
    D = emb.shape[1]
    PL = R * 128
    rows_per_tile = R // _NW

    mesh = plsc.VectorSubcoreMesh(core_axis_name="c", subcore_axis_name="s")

    @functools.partial(
        pl.kernel,
        out_type=jax.ShapeDtypeStruct((PL, D), jnp.float32),
        mesh=mesh,
        scratch_types=[
            pltpu.VMEM((rows_per_tile, 128), jnp.int32),
            pltpu.VMEM((2, 128), jnp.int32),
            pltpu.VMEM((2, 128, D), jnp.float32),
            pltpu.SemaphoreType.DMA,
            pltpu.SemaphoreType.DMA,
            pltpu.SemaphoreType.DMA,
        ],
    )
    def k(idx_hbm, types_hbm, emb_hbm, out_hbm, idx_v, t_v, rows_v, sem1, sem2,
          sem3):
        wid = lax.axis_index("s") * _NC + lax.axis_index("c")
        base_row = wid * rows_per_tile
        pltpu.sync_copy(idx_hbm.at[pl.ds(base_row, rows_per_tile)], idx_v)
        pltpu.async_copy(types_hbm.at[idx_v.at[0]], t_v.at[0], sem1)
        for j in range(rows_per_tile):
            b = j % 2
            pltpu.make_async_copy(types_hbm.at[idx_v.at[j]], t_v.at[b],
                                  sem1).wait()
            if j + 1 < rows_per_tile:
                pltpu.async_copy(types_hbm.at[idx_v.at[j + 1]], t_v.at[1 - b],
                                 sem1)
            if j >= 2:
                pltpu.make_async_copy(
                    rows_v.at[b], out_hbm.at[pl.ds((base_row + j - 2) * 128,
                                                   128)], sem3).wait()
            pltpu.async_copy(emb_hbm.at[t_v.at[b]], rows_v.at[b], sem2)
            pltpu.make_async_copy(emb_hbm.at[t_v.at[b]], rows_v.at[b],
                                  sem2).wait()
            pltpu.async_copy(rows_v.at[b],
                             out_hbm.at[pl.ds((base_row + j) * 128, 128)], sem3)
        for j in range(rows_per_tile - 2, rows_per_tile):
            b = j % 2
            pltpu.make_async_copy(
                rows_v.at[b], out_hbm.at[pl.ds((base_row + j) * 128, 128)],
                sem3).wait()

    return k(idx2, types, emb)


def _tc_encode(occ3, child, vert, maskf, orient_emb, proj_W, proj_b2,
               gru_Wih, gru_Whh, gru_bih2, gru_bhh2):
    P, L, D = occ3.shape
    BP = 512
    grid = (P // BP,)

    def body(occ_ref, child_ref, vert_ref, mask_ref, oemb_ref, projW_ref,
             projb_ref, wih_ref, whh_ref, bih_ref, bhh_ref,
             nodes_out_ref, orient_out_ref):
        oemb = oemb_ref[...]
        wih = wih_ref[...]
        Ao = jnp.dot(jnp.dot(oemb, projW_ref[:D, :],
                             preferred_element_type=jnp.float32), wih,
                     preferred_element_type=jnp.float32)
        Bo = jnp.dot(jnp.dot(oemb, projW_ref[D:, :],
                             preferred_element_type=jnp.float32), wih,
                     preferred_element_type=jnp.float32)
        pbW = jnp.dot(projb_ref[...], wih, preferred_element_type=jnp.float32)
        bih = bih_ref[...]
        whh = whh_ref[...]
        bhh = bhh_ref[...]
        iota = lax.broadcasted_iota(jnp.int32, (1, oemb.shape[0]), 1)

        def gru_step(gx, h):
            gh = jnp.dot(h, whh, preferred_element_type=jnp.float32) + bhh
            z = jax.nn.sigmoid(gx[:, :D] + gh[:, :D])
            r = jax.nn.sigmoid(gx[:, D:2 * D] + gh[:, D:2 * D])
            n = jnp.tanh(gx[:, 2 * D:] + r * gh[:, 2 * D:])
            return (1.0 - z) * n + z * h

        h = jnp.zeros((BP, D), jnp.float32)
        for l in range(L):
            ml = mask_ref[:, l:l + 1]
            gxe = ml * jnp.dot(occ_ref[:, l, :], wih,
                               preferred_element_type=jnp.float32) + bih
            h = gru_step(gxe, h)
            nodes_out_ref[:, l, :] = h * ml
            ohc = (child_ref[:, l:l + 1] == iota).astype(jnp.float32)
            ohv = (vert_ref[:, l:l + 1] == iota).astype(jnp.float32)
            gxo = ml * (jnp.dot(ohc, Ao, preferred_element_type=jnp.float32)
                        + jnp.dot(ohv, Bo, preferred_element_type=jnp.float32)
                        + pbW) + bih
            h = gru_step(gxo, h)
            orient_out_ref[:, l, :] = h * ml

    V = orient_emb.shape[0]
    out = pl.pallas_call(
        body,
        grid=grid,
        in_specs=[
            pl.BlockSpec((BP, L, D), lambda i: (i, 0, 0)),
            pl.BlockSpec((BP, L), lambda i: (i, 0)),
            pl.BlockSpec((BP, L), lambda i: (i, 0)),
            pl.BlockSpec((BP, L), lambda i: (i, 0)),
            pl.BlockSpec((V, D), lambda i: (0, 0)),
            pl.BlockSpec((2 * D, D), lambda i: (0, 0)),
            pl.BlockSpec((1, D), lambda i: (0, 0)),
            pl.BlockSpec((D, 3 * D), lambda i: (0, 0)),
            pl.BlockSpec((D, 3 * D), lambda i: (0, 0)),
            pl.BlockSpec((1, 3 * D), lambda i: (0, 0)),
            pl.BlockSpec((1, 3 * D), lambda i: (0, 0)),
        ],
        out_specs=[
            pl.BlockSpec((BP, L, D), lambda i: (i, 0, 0)),
            pl.BlockSpec((BP, L, D), lambda i: (i, 0, 0)),
        ],
        out_shape=[
            jax.ShapeDtypeStruct((P, L, D), jnp.float32),
            jax.ShapeDtypeStruct((P, L, D), jnp.float32),
        ],
    )(occ3, child, vert, maskf, orient_emb, proj_W, proj_b2,
      gru_Wih, gru_Whh, gru_bih2, gru_bhh2)
    return out


def _sc_scatter(contrib, idx64, n_nodes, d):
    R = idx64.shape[0]
    CH = 32
    RANGE = 12512
    SP_ROWS = RANGE + 32
    idx_rows_per_tile = R // _NS
    n_chunks = idx_rows_per_tile
    ZCH = SP_ROWS // 16
    WCH = RANGE // 16

    mesh = plsc.VectorSubcoreMesh(core_axis_name="c", subcore_axis_name="s")

    @functools.partial(
        pl.kernel,
        out_type=jax.ShapeDtypeStruct((n_nodes, d), jnp.float32),
        mesh=mesh,
        scratch_types=[
            pltpu.VMEM_SHARED((SP_ROWS, d), jnp.float32),
            pltpu.VMEM((16, d), jnp.float32),
            pltpu.VMEM((2, CH), jnp.int32),
            pltpu.VMEM((2, CH), jnp.int32),
            pltpu.VMEM((2, CH, d), jnp.float32),
            pltpu.SemaphoreType.DMA,
            pltpu.SemaphoreType.DMA,
        ],
    )
    def k(contrib_hbm, idx_hbm, out_hbm, spmem, zbuf, ibuf, tbuf, cbuf, semr,
          semi):
        core = lax.axis_index("c")
        sub = lax.axis_index("s")
        trash = RANGE + sub
        zero16 = jnp.zeros((16,), jnp.float32)
        for zr in range(16):
            for zc in range(d // 16):
                zbuf[zr, pl.ds(zc * 16, 16)] = zero16

        for p in range(2):
            lo = (2 * core + p) * RANGE

            def zero_body(kk):
                cid = kk * _NS + sub
                @pl.when(cid < ZCH)
                def _():
                    pltpu.sync_copy(zbuf, spmem.at[pl.ds(cid * 16, 16)])
            pl.loop(0, (ZCH + _NS - 1) // _NS)(zero_body)
            plsc.subcore_barrier()

            pltpu.async_copy(
                contrib_hbm.at[pl.ds(sub * n_chunks * CH, CH)],
                cbuf.at[0], semr)
            pltpu.async_copy(idx_hbm.at[sub * n_chunks], ibuf.at[0], semi)

            def acc_body(j):
                for b in range(2):
                    jj = j + b
                    @pl.when(jj + 1 < n_chunks)
                    def _():
                        pltpu.async_copy(
                            contrib_hbm.at[pl.ds(
                                (sub * n_chunks + jj + 1) * CH, CH)],
                            cbuf.at[1 - b], semr)
                        pltpu.async_copy(idx_hbm.at[sub * n_chunks + jj + 1],
                                         ibuf.at[1 - b], semi)
                    pltpu.make_async_copy(idx_hbm.at[sub * n_chunks + jj],
                                          ibuf.at[b], semi).wait()
                    for kk in range(CH // 16):
                        v = ibuf[b, pl.ds(kk * 16, 16)]
                        inr = (v >= lo) & (v < lo + RANGE)
                        tbuf[b, pl.ds(kk * 16, 16)] = jnp.where(
                            inr, v - lo, trash)
                    pltpu.make_async_copy(
                        contrib_hbm.at[pl.ds((sub * n_chunks + jj) * CH, CH)],
                        cbuf.at[b], semr).wait()
                    pltpu.sync_copy(cbuf.at[b], spmem.at[tbuf.at[b]],
                                    add=True)
            pl.loop(0, n_chunks, step=2)(acc_body)
            plsc.subcore_barrier()

            def wr_body(kk):
                cid = kk * _NS + sub
                s = lo + cid * 16
                @pl.when((cid < WCH) & (s < n_nodes))
                def _():
                    pltpu.sync_copy(spmem.at[pl.ds(cid * 16, 16)],
                                    out_hbm.at[pl.ds(s, 16)])
            pl.loop(0, (WCH + _NS - 1) // _NS)(wr_body)
            plsc.subcore_barrier()

    return k(contrib, idx64)


def kernel(ast_paths_node_indices, ast_paths_lengths, ast_paths_mask,
           ast_nodes_types, ast_paths_child_place, ast_paths_vertical_direction,
           node_type_emb, orient_emb, proj_W, proj_b,
           gru_Wih, gru_Whh, gru_bih, gru_bhh):
    P, L = ast_paths_node_indices.shape
    D = node_type_emb.shape[1]
    N = ast_nodes_types.shape[0]

    idx2 = ast_paths_node_indices.reshape(P * L // 128, 128).astype(jnp.int32)
    types = ast_nodes_types.astype(jnp.int32)
    maskf = ast_paths_mask.astype(jnp.float32)

    node_occ = _sc_gather(idx2, types, node_type_emb)
    occ3 = node_occ.reshape(P, L, D)

    nodes_enc, orient_enc = _tc_encode(
        occ3, ast_paths_child_place.astype(jnp.int32),
        ast_paths_vertical_direction.astype(jnp.int32), maskf,
        orient_emb, proj_W, proj_b.reshape(1, D),
        gru_Wih, gru_Whh, gru_bih.reshape(1, 3 * D), gru_bhh.reshape(1, 3 * D))

    contrib = nodes_enc.reshape(P * L, D)
    idx64 = ast_paths_node_indices.reshape(P * L // 32, 32).astype(jnp.int32)
    node_repr = _sc_scatter(contrib, idx64, N, D)
    return node_repr, nodes_enc, orient_enc
